# baseline (speedup 1.0000x reference)
import functools

import jax
import jax.numpy as jnp
from jax import lax
from jax.experimental import pallas as pl
from jax.experimental.pallas import tpu as pltpu
from jax.experimental.pallas import tpu_sc as plsc

NC, NS = 2, 16
NW = NC * NS
PP = 16


def _sc_gather(xr, pr, idxg, chunk):
    E = idxg[0].shape[0]
    e_per_w = E // NW
    mesh = plsc.VectorSubcoreMesh(core_axis_name="c", subcore_axis_name="s",
                                  num_cores=NC, num_subcores=NS)

    @functools.partial(
        pl.kernel,
        out_type=(jax.ShapeDtypeStruct((E, 128), jnp.float32),
                  jax.ShapeDtypeStruct((E, PP), jnp.float32)),
        mesh=mesh,
        scratch_types=(pltpu.VMEM((chunk,), jnp.int32),
                       pltpu.VMEM((chunk,), jnp.int32),
                       pltpu.VMEM((chunk, 128), jnp.float32),
                       pltpu.VMEM((chunk, PP), jnp.float32),
                       pltpu.SemaphoreType.DMA,
                       pltpu.SemaphoreType.DMA),
        compiler_params=pltpu.CompilerParams(use_tc_tiling_on_sc=False),
    )
    def k(xr_hbm, pr_hbm, idxk_hbm, idxn_hbm, xg_hbm, gp_hbm,
          idxk_v, idxn_v, rows_v, prow_v, sx, sp):
        wid = lax.axis_index("s") * NC + lax.axis_index("c")
        base0 = wid * e_per_w

        def body(j, carry):
            base = base0 + j * chunk
            pltpu.sync_copy(idxk_hbm.at[pl.ds(base, chunk)], idxk_v)
            pltpu.sync_copy(idxn_hbm.at[pl.ds(base, chunk)], idxn_v)
            cx = pltpu.async_copy(xr_hbm.at[idxk_v], rows_v, sx)
            cp = pltpu.async_copy(pr_hbm.at[idxn_v], prow_v, sp)
            cx.wait()
            cp.wait()
            pltpu.sync_copy(rows_v, xg_hbm.at[pl.ds(base, chunk)])
            pltpu.sync_copy(prow_v, gp_hbm.at[pl.ds(base, chunk)])
            return carry

        lax.fori_loop(0, e_per_w // chunk, body, 0)

    return k(xr, pr, idxg[0], idxg[1])


def _tc_body(p_ref, gp_ref, xg_ref, a_ref, w1_ref, b1_ref, w2_ref, b2_ref,
             g1_ref, be1_ref, g2_ref, be2_ref, out_ref):
    B = p_ref.shape[0]
    H1 = w1_ref.shape[1]
    H2 = w2_ref.shape[1]
    C = out_ref.shape[2]
    K = xg_ref.shape[1]
    f32 = jnp.float32

    bf16 = jnp.bfloat16

    y1s = []
    for b in range(B):
        y1 = (jnp.dot(p_ref[b], a_ref[...], preferred_element_type=f32)
              - jnp.dot(gp_ref[b].astype(bf16), w1_ref[...],
                        preferred_element_type=f32)
              + b1_ref[...])
        y1s.append(y1)

    def bn_relu(ys, h, g_ref, be_ref):
        s = sum(jnp.sum(y, axis=1, keepdims=True) for y in ys)
        s2 = sum(jnp.sum(y * y, axis=1, keepdims=True) for y in ys)
        m = s / (B * h)
        var = s2 / (B * h) - m * m
        inv = lax.rsqrt(var + 1e-5)
        scale = inv * g_ref[...]
        shift = be_ref[...] - m * scale
        return [jnp.maximum(y * scale + shift, 0.0) for y in ys]

    y1s = bn_relu(y1s, H1, g1_ref, be1_ref)
    y2s = [jnp.dot(y.astype(bf16), w2_ref[...], preferred_element_type=f32)
           + b2_ref[...]
           for y in y1s]
    y2s = bn_relu(y2s, H2, g2_ref, be2_ref)

    for b in range(B):
        acc = y2s[b][:, 0:C] * xg_ref[b, 0].astype(jnp.float32)
        for k in range(1, K):
            acc = acc + y2s[b][:, k * C:(k + 1) * C] * xg_ref[b, k].astype(jnp.float32)
        out_ref[b] = acc


def _tc_mlp(p16, gpg, xg, AT, W1pT, b1r, W2T, b2r, g1c, be1c, g2c, be2c, TN):
    B, N, _ = p16.shape
    K = xg.shape[1]
    C = xg.shape[3]
    KP = gpg.shape[2]
    H1 = W1pT.shape[1]
    H2 = W2T.shape[1]
    grid = (N // TN,)
    return pl.pallas_call(
        _tc_body,
        grid=grid,
        in_specs=[
            pl.BlockSpec((B, TN, PP), lambda i: (0, i, 0)),
            pl.BlockSpec((B, TN, KP), lambda i: (0, i, 0)),
            pl.BlockSpec((B, K, TN, C), lambda i: (0, 0, i, 0)),
            pl.BlockSpec((PP, H1), lambda i: (0, 0)),
            pl.BlockSpec((KP, H1), lambda i: (0, 0)),
            pl.BlockSpec((1, H1), lambda i: (0, 0)),
            pl.BlockSpec((H1, H2), lambda i: (0, 0)),
            pl.BlockSpec((1, H2), lambda i: (0, 0)),
            pl.BlockSpec((TN, 1), lambda i: (i, 0)),
            pl.BlockSpec((TN, 1), lambda i: (i, 0)),
            pl.BlockSpec((TN, 1), lambda i: (i, 0)),
            pl.BlockSpec((TN, 1), lambda i: (i, 0)),
        ],
        out_specs=pl.BlockSpec((B, TN, C), lambda i: (0, i, 0)),
        out_shape=jax.ShapeDtypeStruct((B, N, C), jnp.float32),
    )(p16, gpg, xg, AT, W1pT, b1r, W2T, b2r, g1c, be1c, g2c, be2c)


def kernel(x, points, W1, b1, g1, beta1, W2, b2, g2, beta2, indices):
    B, N, C = x.shape
    K = indices.shape[2]
    H1 = W1.shape[0]
    H2 = W2.shape[0]

    pr = jnp.pad(points, ((0, 0), (0, 0), (0, PP - 3))).reshape(B * N, PP)
    idxb = (indices.astype(jnp.int32)
            + (jnp.arange(B, dtype=jnp.int32) * N)[:, None, None])

    w1kp = jnp.pad(W1.T.reshape(K, 3, H1), ((0, 0), (0, PP - 3), (0, 0)))
    W1pT = w1kp.reshape(K * PP, H1).astype(jnp.bfloat16)
    AT = jnp.sum(w1kp, axis=0).astype(jnp.bfloat16)
    b1r = b1.reshape(1, H1)
    W2T = W2.T.astype(jnp.bfloat16)
    b2r = b2.reshape(1, H2)
    g1c = g1.reshape(N, 1)
    be1c = beta1.reshape(N, 1)
    g2c = g2.reshape(N, 1)
    be2c = beta2.reshape(N, 1)

    p16b = pr.reshape(B, N, PP).astype(jnp.bfloat16)
    xr = x.reshape(B * N, C)

    slices = ((0, 4800, 600), (4800, 5200, 520))
    gathered = []
    for n0, ns, ck in slices:
        sl = idxb[:, n0:n0 + ns, :]
        idxk = sl.transpose(0, 2, 1).reshape(B * K * ns)
        idxn = sl.reshape(B * K * ns)
        gathered.append(_sc_gather(xr, pr, (idxk, idxn), chunk=ck))

    outs = []
    for (n0, ns, ck), (xg, gpp) in zip(slices, gathered):
        if outs:
            xg, gpp, prev = lax.optimization_barrier((xg, gpp, outs[-1]))
            outs[-1] = prev
        xg4 = xg.reshape(B, K, ns, C)
        gpr = gpp.reshape(B, ns, K * PP)
        outs.append(_tc_mlp(p16b[:, n0:n0 + ns], gpr, xg4, AT, W1pT, b1r,
                            W2T, b2r, g1c[n0:n0 + ns], be1c[n0:n0 + ns],
                            g2c[n0:n0 + ns], be2c[n0:n0 + ns], TN=400))
    out = jnp.concatenate(outs, axis=1)
    return (out, points, indices)


# speedup vs baseline: 1.3043x; 1.3043x over previous
import functools

import jax
import jax.numpy as jnp
from jax import lax
from jax.experimental import pallas as pl
from jax.experimental.pallas import tpu as pltpu
from jax.experimental.pallas import tpu_sc as plsc

NC, NS = 2, 16
NW = NC * NS
PP = 16


def _sc_gather(xr, pr, idx2, chunk):
    E = idx2.shape[1]
    e_per_w = E // NW
    niters = e_per_w // chunk
    pairs = niters // 2
    tail = niters % 2
    mesh = plsc.VectorSubcoreMesh(core_axis_name="c", subcore_axis_name="s",
                                  num_cores=NC, num_subcores=NS)

    @functools.partial(
        pl.kernel,
        out_type=(jax.ShapeDtypeStruct((E, 128), jnp.float32),
                  jax.ShapeDtypeStruct((E, PP), jnp.float32)),
        mesh=mesh,
        scratch_types=(pltpu.VMEM((2, chunk), jnp.int32),
                       pltpu.VMEM((2, chunk), jnp.int32),
                       pltpu.VMEM((chunk, 128), jnp.float32),
                       pltpu.VMEM((chunk, 128), jnp.float32),
                       pltpu.VMEM((chunk, PP), jnp.float32),
                       pltpu.VMEM((chunk, PP), jnp.float32),
                       pltpu.SemaphoreType.DMA,
                       pltpu.SemaphoreType.DMA),
        compiler_params=pltpu.CompilerParams(use_tc_tiling_on_sc=False),
    )
    def k(xr_hbm, pr_hbm, idx2_hbm, xg_hbm, gp_hbm,
          idx_v0, idx_v1, rows0, rows1, prow0, prow1, sg0, sg1):
        wid = lax.axis_index("s") * NC + lax.axis_index("c")
        base0 = wid * e_per_w

        def load_idx(j, dst):
            pltpu.sync_copy(idx2_hbm.at[:, pl.ds(base0 + j * chunk, chunk)], dst)

        def start_g(idxv, rows, prow, sem):
            pltpu.async_copy(xr_hbm.at[idxv.at[0]], rows, sem)
            pltpu.async_copy(pr_hbm.at[idxv.at[1]], prow, sem)

        def wait_g(idxv, rows, prow, sem):
            pltpu.make_async_copy(xr_hbm.at[idxv.at[0]], rows, sem).wait()
            pltpu.make_async_copy(pr_hbm.at[idxv.at[1]], prow, sem).wait()

        def write(j, rows, prow):
            base = base0 + j * chunk
            pltpu.sync_copy(rows, xg_hbm.at[pl.ds(base, chunk)])
            pltpu.sync_copy(prow, gp_hbm.at[pl.ds(base, chunk)])

        load_idx(0, idx_v0)
        start_g(idx_v0, rows0, prow0, sg0)

        def body(i, carry):
            c0 = 2 * i
            load_idx(c0 + 1, idx_v1)
            wait_g(idx_v0, rows0, prow0, sg0)
            start_g(idx_v1, rows1, prow1, sg1)
            write(c0, rows0, prow0)

            @pl.when(c0 + 2 < niters)
            def _():
                load_idx(c0 + 2, idx_v0)
                start_g(idx_v0, rows0, prow0, sg0)

            wait_g(idx_v1, rows1, prow1, sg1)
            write(c0 + 1, rows1, prow1)
            return carry

        lax.fori_loop(0, pairs, body, 0)
        if tail:
            wait_g(idx_v0, rows0, prow0, sg0)
            write(niters - 1, rows0, prow0)

    return k(xr, pr, idx2)


def _tc_body(p_ref, gp_ref, xg_ref, a_ref, w1_ref, b1_ref, w2_ref, b2_ref,
             g1_ref, be1_ref, g2_ref, be2_ref, out_ref):
    B = p_ref.shape[0]
    H1 = w1_ref.shape[1]
    H2 = w2_ref.shape[1]
    C = out_ref.shape[2]
    K = xg_ref.shape[1]
    f32 = jnp.float32
    bf16 = jnp.bfloat16

    y1s = []
    for b in range(B):
        y1 = (jnp.dot(p_ref[b], a_ref[...], preferred_element_type=f32)
              - jnp.dot(gp_ref[b].astype(bf16), w1_ref[...],
                        preferred_element_type=f32)
              + b1_ref[...])
        y1s.append(y1)

    def bn_relu(ys, h, g_ref, be_ref):
        s = sum(jnp.sum(y, axis=1, keepdims=True) for y in ys)
        s2 = sum(jnp.sum(y * y, axis=1, keepdims=True) for y in ys)
        m = s / (B * h)
        var = s2 / (B * h) - m * m
        inv = lax.rsqrt(var + 1e-5)
        scale = inv * g_ref[...]
        shift = be_ref[...] - m * scale
        return [jnp.maximum(y * scale + shift, 0.0) for y in ys]

    y1s = bn_relu(y1s, H1, g1_ref, be1_ref)
    y2s = [jnp.dot(y.astype(bf16), w2_ref[...], preferred_element_type=f32)
           + b2_ref[...]
           for y in y1s]
    y2s = bn_relu(y2s, H2, g2_ref, be2_ref)

    for b in range(B):
        acc = y2s[b][:, 0:C] * xg_ref[b, 0]
        for k in range(1, K):
            acc = acc + y2s[b][:, k * C:(k + 1) * C] * xg_ref[b, k]
        out_ref[b] = acc


def _tc_mlp(p16, gpg, xg, AT, W1pT, b1r, W2T, b2r, g1c, be1c, g2c, be2c, TN):
    B, N, _ = p16.shape
    K = xg.shape[1]
    C = xg.shape[3]
    KP = gpg.shape[2]
    H1 = W1pT.shape[1]
    H2 = W2T.shape[1]
    grid = (N // TN,)
    return pl.pallas_call(
        _tc_body,
        grid=grid,
        in_specs=[
            pl.BlockSpec((B, TN, PP), lambda i: (0, i, 0)),
            pl.BlockSpec((B, TN, KP), lambda i: (0, i, 0)),
            pl.BlockSpec((B, K, TN, C), lambda i: (0, 0, i, 0)),
            pl.BlockSpec((PP, H1), lambda i: (0, 0)),
            pl.BlockSpec((KP, H1), lambda i: (0, 0)),
            pl.BlockSpec((1, H1), lambda i: (0, 0)),
            pl.BlockSpec((H1, H2), lambda i: (0, 0)),
            pl.BlockSpec((1, H2), lambda i: (0, 0)),
            pl.BlockSpec((TN, 1), lambda i: (i, 0)),
            pl.BlockSpec((TN, 1), lambda i: (i, 0)),
            pl.BlockSpec((TN, 1), lambda i: (i, 0)),
            pl.BlockSpec((TN, 1), lambda i: (i, 0)),
        ],
        out_specs=pl.BlockSpec((B, TN, C), lambda i: (0, i, 0)),
        out_shape=jax.ShapeDtypeStruct((B, N, C), jnp.float32),
    )(p16, gpg, xg, AT, W1pT, b1r, W2T, b2r, g1c, be1c, g2c, be2c)


def kernel(x, points, W1, b1, g1, beta1, W2, b2, g2, beta2, indices):
    B, N, C = x.shape
    K = indices.shape[2]
    H1 = W1.shape[0]
    H2 = W2.shape[0]

    pr = jnp.pad(points, ((0, 0), (0, 0), (0, PP - 3))).reshape(B * N, PP)
    idxb = (indices.astype(jnp.int32)
            + (jnp.arange(B, dtype=jnp.int32) * N)[:, None, None])
    idxk = idxb.transpose(0, 2, 1).reshape(B * N * K)
    idxn = idxb.reshape(B * N * K)
    idx2 = jnp.stack([idxk, idxn])

    w1kp = jnp.pad(W1.T.reshape(K, 3, H1), ((0, 0), (0, PP - 3), (0, 0)))
    W1pT = w1kp.reshape(K * PP, H1).astype(jnp.bfloat16)
    AT = jnp.sum(w1kp, axis=0).astype(jnp.bfloat16)
    b1r = b1.reshape(1, H1)
    W2T = W2.T.astype(jnp.bfloat16)
    b2r = b2.reshape(1, H2)
    g1c = g1.reshape(N, 1)
    be1c = beta1.reshape(N, 1)
    g2c = g2.reshape(N, 1)
    be2c = beta2.reshape(N, 1)

    xg, gpp = _sc_gather(x.reshape(B * N, C), pr, idx2, chunk=400)
    xg4 = xg.reshape(B, K, N, C)
    gpr = gpp.reshape(B, N, K * PP)

    p16b = pr.reshape(B, N, PP).astype(jnp.bfloat16)
    out = _tc_mlp(p16b, gpr, xg4, AT, W1pT, b1r, W2T, b2r,
                  g1c, be1c, g2c, be2c, TN=400)
    return (out, points, indices)
